# Initial kernel scaffold; baseline (speedup 1.0000x reference)
import jax
import jax.numpy as jnp
from jax.experimental import pallas as pl


def kernel(x, edge_index, edge_weight, Wi1, bi1, gi1, bti1, Wi2, bi2, gi2, bti2, Wg1, bg1, gg1, btg1, Wg2, bg2, gg2, btg2, Wo1, bo1, go1, bto1, Wo2, bo2, go2, bto2, Wo3, bo3):
    raise NotImplementedError("write your pallas kernel here")


# speedup vs baseline: 6.4064x; 6.4064x over previous
import functools

import jax
import jax.numpy as jnp
from jax import lax
from jax.experimental import pallas as pl
from jax.experimental.pallas import tpu as pltpu
from jax.experimental.pallas import tpu_sc as plsc

N = 10000
E = 320000
D = 128
L = 16
NC = 2
NS = 16
NW = NC * NS
EB = 128
NBATCH = 80
ET = EB * NBATCH
E_PAD = ET * NW
N_PAD = 10240
ROWS_PER_TILE = N_PAD // NS
ZROWS = 128

_MESH = plsc.VectorSubcoreMesh(core_axis_name="c", subcore_axis_name="s")


def _wid():
    return lax.axis_index("c") * NS + lax.axis_index("s")


def _deg_body(col_hbm, ew_hbm, out_hbm, colv, ewv, acc):
    wid = _wid()

    def zero(i, _):
        acc[pl.ds(i * L, L)] = jnp.zeros((L,), jnp.float32)
        return _

    lax.fori_loop(0, N_PAD // L, zero, None)
    base = wid * ET
    pltpu.sync_copy(col_hbm.at[pl.ds(base, ET)], colv)
    pltpu.sync_copy(ew_hbm.at[pl.ds(base, ET)], ewv)

    def body(i, _):
        idx = colv[pl.ds(i * L, L)]
        w = ewv[pl.ds(i * L, L)]
        plsc.addupdate_scatter(acc, [idx], w)
        return _

    lax.fori_loop(0, ET // L, body, None)
    pltpu.sync_copy(acc, out_hbm.at[wid])


@jax.jit
def _deg(col, ew):
    return pl.kernel(
        _deg_body,
        out_type=jax.ShapeDtypeStruct((NW, N_PAD), jnp.float32),
        mesh=_MESH,
        scratch_types=[
            pltpu.VMEM((ET,), jnp.int32),
            pltpu.VMEM((ET,), jnp.float32),
            pltpu.VMEM((N_PAD,), jnp.float32),
        ],
        compiler_params=pltpu.CompilerParams(needs_layout_passes=False),
    )(col, ew)


def _spmm_body(h3_hbm, row_hbm, col_hbm, ew_hbm, out_hbm,
               ridx, cidx, ewb, rows, acc, gsem):
    cid = lax.axis_index("c")
    sid = lax.axis_index("s")
    wid = cid * NS + sid

    def zfill(i, _):
        for g in range(D // L):
            rows[i, pl.ds(g * L, L)] = jnp.zeros((L,), jnp.float32)
        return _

    lax.fori_loop(0, EB, zfill, None)
    for j in range(ROWS_PER_TILE // EB):
        pltpu.sync_copy(rows, acc.at[pl.ds(sid * ROWS_PER_TILE + j * EB, EB)])

    plsc.subcore_barrier()

    def batch(b, _):
        base = wid * ET + b * EB
        pltpu.sync_copy(row_hbm.at[pl.ds(base, EB)], ridx)
        pltpu.sync_copy(col_hbm.at[pl.ds(base, EB)], cidx)
        pltpu.sync_copy(ew_hbm.at[pl.ds(base, EB)], ewb)
        pltpu.async_copy(h3_hbm.at[ridx], rows, gsem).wait()

        def scale(j, _s):
            ew16 = ewb[pl.ds(j * L, L)]
            for k in range(L):
                sv = jnp.full((L,), ew16[k], jnp.float32)
                i = j * L + k
                for g in range(D // L):
                    rows[i, pl.ds(g * L, L)] = rows[i, pl.ds(g * L, L)] * sv
            return _s

        lax.fori_loop(0, EB // L, scale, None)
        pltpu.sync_copy(rows, acc.at[cidx], add=True)
        return _

    lax.fori_loop(0, NBATCH, batch, None)

    plsc.subcore_barrier()
    pltpu.sync_copy(acc.at[pl.ds(sid * ROWS_PER_TILE, ROWS_PER_TILE)],
                    out_hbm.at[cid, pl.ds(sid * ROWS_PER_TILE, ROWS_PER_TILE)])


@jax.jit
def _spmm(h3, row, col, ew):
    return pl.kernel(
        _spmm_body,
        out_type=jax.ShapeDtypeStruct((NC, N_PAD, D), jnp.float32),
        mesh=_MESH,
        scratch_types=[
            pltpu.VMEM((EB,), jnp.int32),
            pltpu.VMEM((EB,), jnp.int32),
            pltpu.VMEM((EB,), jnp.float32),
            pltpu.VMEM((EB, D), jnp.float32),
            pltpu.VMEM_SHARED((N_PAD, D), jnp.float32),
            pltpu.SemaphoreType.DMA,
        ],
    )(h3, row, col, ew)


def _bn_relu(h, g, b):
    m = jnp.mean(h, axis=0, keepdims=True)
    v = jnp.mean(h * h, axis=0, keepdims=True) - m * m
    return jax.nn.relu((h - m) * lax.rsqrt(v + 1e-5) * g + b)


def _dinv_col(dp_ref):
    deg = jnp.sum(dp_ref[...], axis=0)[:N] + 1.0
    return lax.rsqrt(deg)[:, None]


def _tc_a_body(x_ref, dp_ref, wi1, bi1, gi1, bti1, wi2, bi2, gi2, bti2, wg1,
               h3a_ref):
    h = jnp.dot(x_ref[...], wi1[...], preferred_element_type=jnp.float32) + bi1[...]
    h = _bn_relu(h, gi1[...], bti1[...])
    h = jnp.dot(h, wi2[...], preferred_element_type=jnp.float32) + bi2[...]
    h = _bn_relu(h, gi2[...], bti2[...])
    t = jnp.dot(h, wg1[...], preferred_element_type=jnp.float32)
    h3a_ref[...] = _dinv_col(dp_ref) * t


def _tc_b_body(h3_ref, p_ref, dp_ref, gg, btg, bg, wnext, h3b_ref):
    dinv = _dinv_col(dp_ref)
    s = dinv * (h3_ref[...] + p_ref[0, :N] + p_ref[1, :N]) + bg[...]
    g = _bn_relu(s, gg[...], btg[...])
    t = jnp.dot(g, wnext[...], preferred_element_type=jnp.float32)
    h3b_ref[...] = dinv * t


def _tc_c_body(h3_ref, p_ref, dp_ref, gg, btg, bg,
               wo1, bo1, go1, bto1, wo2, bo2, go2, bto2, wo3, bo3, out_ref):
    dinv = _dinv_col(dp_ref)
    s = dinv * (h3_ref[...] + p_ref[0, :N] + p_ref[1, :N]) + bg[...]
    g = _bn_relu(s, gg[...], btg[...])
    o = _bn_relu(jnp.dot(g, wo1[...], preferred_element_type=jnp.float32) + bo1[...],
                 go1[...], bto1[...])
    o = _bn_relu(jnp.dot(o, wo2[...], preferred_element_type=jnp.float32) + bo2[...],
                 go2[...], bto2[...])
    out_ref[...] = jnp.dot(o, wo3[...], preferred_element_type=jnp.float32) + bo3[...]


def _tc_call(body, out_shape, *args):
    return pl.pallas_call(
        body,
        out_shape=jax.ShapeDtypeStruct(out_shape, jnp.float32),
    )(*args)


def kernel(x, edge_index, edge_weight,
           Wi1, bi1, gi1, bti1, Wi2, bi2, gi2, bti2,
           Wg1, bg1, gg1, btg1, Wg2, bg2, gg2, btg2,
           Wo1, bo1, go1, bto1, Wo2, bo2, go2, bto2, Wo3, bo3):
    pad = E_PAD - E
    rowp = jnp.concatenate([edge_index[0], jnp.zeros((pad,), jnp.int32)])
    colp = jnp.concatenate([edge_index[1], jnp.zeros((pad,), jnp.int32)])
    ewp = jnp.concatenate([edge_weight, jnp.zeros((pad,), jnp.float32)])

    dp = _deg(colp, ewp)

    h3a = _tc_call(_tc_a_body, (N, D),
                   x, dp, Wi1, bi1, gi1, bti1, Wi2, bi2, gi2, bti2, Wg1)
    p1 = _spmm(h3a, rowp, colp, ewp)
    h3b = _tc_call(_tc_b_body, (N, D), h3a, p1, dp, gg1, btg1, bg1, Wg2)
    p2 = _spmm(h3b, rowp, colp, ewp)
    out = _tc_call(_tc_c_body, (N, 4), h3b, p2, dp, gg2, btg2, bg2,
                   Wo1, bo1, go1, bto1, Wo2, bo2, go2, bto2, Wo3, bo3)
    return out


# speedup vs baseline: 7.8752x; 1.2293x over previous
import functools

import jax
import jax.numpy as jnp
from jax import lax
from jax.experimental import pallas as pl
from jax.experimental.pallas import tpu as pltpu
from jax.experimental.pallas import tpu_sc as plsc

N = 10000
E = 320000
D = 128
L = 16
NC = 2
NS = 16
NW = NC * NS
EB = 128
NBATCH = 80
ET = EB * NBATCH
E_PAD = ET * NW
N_PAD = 10240
ROWS_PER_TILE = N_PAD // NS
ZROWS = 128

_MESH = plsc.VectorSubcoreMesh(core_axis_name="c", subcore_axis_name="s")


def _wid():
    return lax.axis_index("c") * NS + lax.axis_index("s")


def _deg_body(col_hbm, ew_hbm, out_hbm, colv, ewv, acc):
    wid = _wid()

    def zero(i, _):
        acc[pl.ds(i * L, L)] = jnp.zeros((L,), jnp.float32)
        return _

    lax.fori_loop(0, N_PAD // L, zero, None)
    base = wid * ET
    pltpu.sync_copy(col_hbm.at[pl.ds(base, ET)], colv)
    pltpu.sync_copy(ew_hbm.at[pl.ds(base, ET)], ewv)

    def body(i, _):
        idx = colv[pl.ds(i * L, L)]
        w = ewv[pl.ds(i * L, L)]
        plsc.addupdate_scatter(acc, [idx], w)
        return _

    lax.fori_loop(0, ET // L, body, None)
    pltpu.sync_copy(acc, out_hbm.at[wid])


@jax.jit
def _deg(col, ew):
    return pl.kernel(
        _deg_body,
        out_type=jax.ShapeDtypeStruct((NW, N_PAD), jnp.float32),
        mesh=_MESH,
        scratch_types=[
            pltpu.VMEM((ET,), jnp.int32),
            pltpu.VMEM((ET,), jnp.float32),
            pltpu.VMEM((N_PAD,), jnp.float32),
        ],
        compiler_params=pltpu.CompilerParams(needs_layout_passes=False),
    )(col, ew)


def _scale_rows(rowsbuf, ebuf):
    def scale(j, _s):
        ew16 = plsc.bitcast(ebuf[2, pl.ds(j * L, L)], jnp.float32)
        for k in range(L):
            sv = jnp.full((L,), ew16[k], jnp.float32)
            i = j * L + k
            for g in range(D // L):
                rowsbuf[i, pl.ds(g * L, L)] = rowsbuf[i, pl.ds(g * L, L)] * sv
        return _s

    lax.fori_loop(0, EB // L, scale, None)


def _spmm_body(h3_hbm, pk_hbm, out_hbm,
               ebuf0, ebuf1, ebuf2, ebuf3, rows0, rows1, acc,
               ge0, ge1, se0, se1, ee0, ee1, ee2, ee3):
    cid = lax.axis_index("c")
    sid = lax.axis_index("s")
    wid = cid * NS + sid
    wb = wid * NBATCH
    ebufs = (ebuf0, ebuf1, ebuf2, ebuf3)
    ees = (ee0, ee1, ee2, ee3)

    def zfill(i, _):
        for g in range(D // L):
            rows0[i, pl.ds(g * L, L)] = jnp.zeros((L,), jnp.float32)
        return _

    lax.fori_loop(0, EB, zfill, None)
    for j in range(ROWS_PER_TILE // EB):
        pltpu.sync_copy(rows0, acc.at[pl.ds(sid * ROWS_PER_TILE + j * EB, EB)])

    plsc.subcore_barrier()

    pltpu.async_copy(pk_hbm.at[wb + 0], ebuf0, ee0)
    pltpu.async_copy(pk_hbm.at[wb + 1], ebuf1, ee1)
    pltpu.async_copy(pk_hbm.at[wb + 2], ebuf2, ee2)
    pltpu.make_async_copy(pk_hbm.at[wb + 0], ebuf0, ee0).wait()
    pltpu.async_copy(h3_hbm.at[ebuf0.at[0]], rows0, ge0)

    def half(b, e, rows_a, rows_b, ge_a, ge_b, se_a, se_b):
        ep = (e + 3) % 4
        en = (e + 1) % 4
        pltpu.make_async_copy(h3_hbm.at[ebufs[e].at[0]], rows_a, ge_a).wait()
        _scale_rows(rows_a, ebufs[e])
        pltpu.async_copy(rows_a, acc.at[ebufs[e].at[1]], se_a, add=True)

        @pl.when(b >= 1)
        def _():
            pltpu.make_async_copy(rows_b, acc.at[ebufs[ep].at[1]], se_b).wait()

        @pl.when(b + 3 < NBATCH)
        def _():
            pltpu.async_copy(pk_hbm.at[wb + b + 3], ebufs[ep], ees[ep])

        @pl.when(b + 1 < NBATCH)
        def _():
            pltpu.make_async_copy(pk_hbm.at[wb + b + 1], ebufs[en], ees[en]).wait()
            pltpu.async_copy(h3_hbm.at[ebufs[en].at[0]], rows_b, ge_b)

    def quad(i, _):
        b = 4 * i
        half(b + 0, 0, rows0, rows1, ge0, ge1, se0, se1)
        half(b + 1, 1, rows1, rows0, ge1, ge0, se1, se0)
        half(b + 2, 2, rows0, rows1, ge0, ge1, se0, se1)
        half(b + 3, 3, rows1, rows0, ge1, ge0, se1, se0)
        return _

    lax.fori_loop(0, NBATCH // 4, quad, None)
    pltpu.make_async_copy(rows1, acc.at[ebuf3.at[1]], se1).wait()

    plsc.subcore_barrier()
    pltpu.sync_copy(acc.at[pl.ds(sid * ROWS_PER_TILE, ROWS_PER_TILE)],
                    out_hbm.at[cid, pl.ds(sid * ROWS_PER_TILE, ROWS_PER_TILE)])


@jax.jit
def _spmm(h3, pk):
    return pl.kernel(
        _spmm_body,
        out_type=jax.ShapeDtypeStruct((NC, N_PAD, D), jnp.float32),
        mesh=_MESH,
        scratch_types=(
            [pltpu.VMEM((8, EB), jnp.int32)] * 4
            + [pltpu.VMEM((EB, D), jnp.float32)] * 2
            + [pltpu.VMEM_SHARED((N_PAD, D), jnp.float32)]
            + [pltpu.SemaphoreType.DMA] * 8
        ),
        compiler_params=pltpu.CompilerParams(needs_layout_passes=False),
    )(h3, pk)


def _bn_relu(h, g, b):
    m = jnp.mean(h, axis=0, keepdims=True)
    v = jnp.mean(h * h, axis=0, keepdims=True) - m * m
    return jax.nn.relu((h - m) * lax.rsqrt(v + 1e-5) * g + b)


def _dinv_col(dp_ref):
    deg = jnp.sum(dp_ref[...], axis=0)[:N] + 1.0
    return lax.rsqrt(deg)[:, None]


def _tc_a_body(x_ref, dp_ref, wi1, bi1, gi1, bti1, wi2, bi2, gi2, bti2, wg1,
               h3a_ref):
    h = jnp.dot(x_ref[...], wi1[...], preferred_element_type=jnp.float32) + bi1[...]
    h = _bn_relu(h, gi1[...], bti1[...])
    h = jnp.dot(h, wi2[...], preferred_element_type=jnp.float32) + bi2[...]
    h = _bn_relu(h, gi2[...], bti2[...])
    t = jnp.dot(h, wg1[...], preferred_element_type=jnp.float32)
    h3a_ref[...] = _dinv_col(dp_ref) * t


def _tc_b_body(h3_ref, p_ref, dp_ref, gg, btg, bg, wnext, h3b_ref):
    dinv = _dinv_col(dp_ref)
    s = dinv * (h3_ref[...] + p_ref[0, :N] + p_ref[1, :N]) + bg[...]
    g = _bn_relu(s, gg[...], btg[...])
    t = jnp.dot(g, wnext[...], preferred_element_type=jnp.float32)
    h3b_ref[...] = dinv * t


def _tc_c_body(h3_ref, p_ref, dp_ref, gg, btg, bg,
               wo1, bo1, go1, bto1, wo2, bo2, go2, bto2, wo3, bo3, out_ref):
    dinv = _dinv_col(dp_ref)
    s = dinv * (h3_ref[...] + p_ref[0, :N] + p_ref[1, :N]) + bg[...]
    g = _bn_relu(s, gg[...], btg[...])
    o = _bn_relu(jnp.dot(g, wo1[...], preferred_element_type=jnp.float32) + bo1[...],
                 go1[...], bto1[...])
    o = _bn_relu(jnp.dot(o, wo2[...], preferred_element_type=jnp.float32) + bo2[...],
                 go2[...], bto2[...])
    out_ref[...] = jnp.dot(o, wo3[...], preferred_element_type=jnp.float32) + bo3[...]


def _tc_call(body, out_shape, *args):
    return pl.pallas_call(
        body,
        out_shape=jax.ShapeDtypeStruct(out_shape, jnp.float32),
    )(*args)


def kernel(x, edge_index, edge_weight,
           Wi1, bi1, gi1, bti1, Wi2, bi2, gi2, bti2,
           Wg1, bg1, gg1, btg1, Wg2, bg2, gg2, btg2,
           Wo1, bo1, go1, bto1, Wo2, bo2, go2, bto2, Wo3, bo3):
    pad = E_PAD - E
    rowp = jnp.concatenate([edge_index[0], jnp.zeros((pad,), jnp.int32)])
    colp = jnp.concatenate([edge_index[1], jnp.zeros((pad,), jnp.int32)])
    ewp = jnp.concatenate([edge_weight, jnp.zeros((pad,), jnp.float32)])

    nblk = E_PAD // EB
    pk = jnp.concatenate([
        rowp.reshape(nblk, 1, EB),
        colp.reshape(nblk, 1, EB),
        lax.bitcast_convert_type(ewp, jnp.int32).reshape(nblk, 1, EB),
        jnp.zeros((nblk, 5, EB), jnp.int32),
    ], axis=1)

    dp = _deg(colp, ewp)

    h3a = _tc_call(_tc_a_body, (N, D),
                   x, dp, Wi1, bi1, gi1, bti1, Wi2, bi2, gi2, bti2, Wg1)
    p1 = _spmm(h3a, pk)
    h3b = _tc_call(_tc_b_body, (N, D), h3a, p1, dp, gg1, btg1, bg1, Wg2)
    p2 = _spmm(h3b, pk)
    out = _tc_call(_tc_c_body, (N, 4), h3b, p2, dp, gg2, btg2, bg2,
                   Wo1, bo1, go1, bto1, Wo2, bo2, go2, bto2, Wo3, bo3)
    return out


# speedup vs baseline: 9.2452x; 1.1740x over previous
import functools

import jax
import jax.numpy as jnp
from jax import lax
from jax.experimental import pallas as pl
from jax.experimental.pallas import tpu as pltpu
from jax.experimental.pallas import tpu_sc as plsc

N = 10000
E = 320000
D = 128
L = 16
NC = 2
NS = 16
NW = NC * NS
EB = 128
NBATCH = 80
NB_A = 120
NB_B = 40
ET = EB * NBATCH
E_PAD = ET * NW
N_PAD = 10240
ROWS_PER_TILE = N_PAD // NS
ZROWS = 128

_MESH = plsc.VectorSubcoreMesh(core_axis_name="c", subcore_axis_name="s")


def _wid():
    return lax.axis_index("c") * NS + lax.axis_index("s")


def _deg_body(col_hbm, ew_hbm, out_hbm, colv, ewv, acc):
    wid = _wid()

    def zero(i, _):
        acc[pl.ds(i * L, L)] = jnp.zeros((L,), jnp.float32)
        return _

    lax.fori_loop(0, N_PAD // L, zero, None)
    base = wid * ET
    pltpu.sync_copy(col_hbm.at[pl.ds(base, ET)], colv)
    pltpu.sync_copy(ew_hbm.at[pl.ds(base, ET)], ewv)

    def body(i, _):
        idx = colv[pl.ds(i * L, L)]
        w = ewv[pl.ds(i * L, L)]
        plsc.addupdate_scatter(acc, [idx], w)
        return _

    lax.fori_loop(0, ET // L, body, None)
    pltpu.sync_copy(acc, out_hbm.at[wid])


@jax.jit
def _deg(col, ew):
    return pl.kernel(
        _deg_body,
        out_type=jax.ShapeDtypeStruct((NW, N_PAD), jnp.float32),
        mesh=_MESH,
        scratch_types=[
            pltpu.VMEM((ET,), jnp.int32),
            pltpu.VMEM((ET,), jnp.float32),
            pltpu.VMEM((N_PAD,), jnp.float32),
        ],
        compiler_params=pltpu.CompilerParams(needs_layout_passes=False),
    )(col, ew)


def _scale_rows(rowsbuf, ebuf):
    def scale(j, _s):
        ew16 = plsc.bitcast(ebuf[2, pl.ds(j * L, L)], jnp.float32)
        for k in range(L):
            sv = jnp.full((L,), ew16[k], jnp.float32)
            i = j * L + k
            for g in range(D // L):
                rowsbuf[i, pl.ds(g * L, L)] = rowsbuf[i, pl.ds(g * L, L)] * sv
        return _s

    lax.fori_loop(0, EB // L, scale, None)


def _spmm_body(h3_hbm, pk_hbm, out_hbm,
               ebuf0, ebuf1, ebuf2, ebuf3, rows0, rows1, acc,
               ge0, ge1, se0, se1, ee0, ee1, ee2, ee3):
    cid = lax.axis_index("c")
    sid = lax.axis_index("s")
    nb = jnp.where(cid == 0, NB_A, NB_B)
    wb = jnp.where(cid == 0, sid * NB_A, NS * NB_A + sid * NB_B)
    ebufs = (ebuf0, ebuf1, ebuf2, ebuf3)
    ees = (ee0, ee1, ee2, ee3)

    def zfill(i, _):
        for g in range(D // L):
            rows0[i, pl.ds(g * L, L)] = jnp.zeros((L,), jnp.float32)
        return _

    lax.fori_loop(0, EB, zfill, None)
    for j in range(ROWS_PER_TILE // EB):
        pltpu.sync_copy(rows0, acc.at[pl.ds(sid * ROWS_PER_TILE + j * EB, EB)])

    plsc.subcore_barrier()

    pltpu.async_copy(pk_hbm.at[wb + 0], ebuf0, ee0)
    pltpu.async_copy(pk_hbm.at[wb + 1], ebuf1, ee1)
    pltpu.async_copy(pk_hbm.at[wb + 2], ebuf2, ee2)
    pltpu.make_async_copy(pk_hbm.at[wb + 0], ebuf0, ee0).wait()
    pltpu.async_copy(h3_hbm.at[ebuf0.at[0]], rows0, ge0)

    def half(b, e, rows_a, rows_b, ge_a, ge_b, se_a, se_b):
        ep = (e + 3) % 4
        en = (e + 1) % 4
        pltpu.make_async_copy(h3_hbm.at[ebufs[e].at[0]], rows_a, ge_a).wait()
        _scale_rows(rows_a, ebufs[e])
        pltpu.async_copy(rows_a, acc.at[ebufs[e].at[1]], se_a, add=True)

        @pl.when(b >= 1)
        def _():
            pltpu.make_async_copy(rows_b, acc.at[ebufs[ep].at[1]], se_b).wait()

        @pl.when(b + 3 < nb)
        def _():
            pltpu.async_copy(pk_hbm.at[wb + b + 3], ebufs[ep], ees[ep])

        @pl.when(b + 1 < nb)
        def _():
            pltpu.make_async_copy(pk_hbm.at[wb + b + 1], ebufs[en], ees[en]).wait()
            pltpu.async_copy(h3_hbm.at[ebufs[en].at[0]], rows_b, ge_b)

    def quad(i, _):
        b = 4 * i
        half(b + 0, 0, rows0, rows1, ge0, ge1, se0, se1)
        half(b + 1, 1, rows1, rows0, ge1, ge0, se1, se0)
        half(b + 2, 2, rows0, rows1, ge0, ge1, se0, se1)
        half(b + 3, 3, rows1, rows0, ge1, ge0, se1, se0)
        return _

    lax.fori_loop(0, nb // 4, quad, None)
    pltpu.make_async_copy(rows1, acc.at[ebuf3.at[1]], se1).wait()

    plsc.subcore_barrier()
    pltpu.sync_copy(acc.at[pl.ds(sid * ROWS_PER_TILE, ROWS_PER_TILE)],
                    out_hbm.at[cid, pl.ds(sid * ROWS_PER_TILE, ROWS_PER_TILE)])


@jax.jit
def _spmm(h3, pk):
    return pl.kernel(
        _spmm_body,
        out_type=jax.ShapeDtypeStruct((NC, N_PAD, D), jnp.float32),
        mesh=_MESH,
        scratch_types=(
            [pltpu.VMEM((8, EB), jnp.int32)] * 4
            + [pltpu.VMEM((EB, D), jnp.float32)] * 2
            + [pltpu.VMEM_SHARED((N_PAD, D), jnp.float32)]
            + [pltpu.SemaphoreType.DMA] * 8
        ),
        compiler_params=pltpu.CompilerParams(needs_layout_passes=False),
    )(h3, pk)


def _bn_relu(h, g, b):
    m = jnp.mean(h, axis=0, keepdims=True)
    v = jnp.mean(h * h, axis=0, keepdims=True) - m * m
    return jax.nn.relu((h - m) * lax.rsqrt(v + 1e-5) * g + b)


def _dinv_col(dp_ref):
    deg = jnp.sum(dp_ref[...], axis=0)[:N] + 1.0
    return lax.rsqrt(deg)[:, None]


def _tc_a_body(x_ref, dp_ref, wi1, bi1, gi1, bti1, wi2, bi2, gi2, bti2, wg1,
               h3a_ref):
    h = jnp.dot(x_ref[...], wi1[...], preferred_element_type=jnp.float32) + bi1[...]
    h = _bn_relu(h, gi1[...], bti1[...])
    h = jnp.dot(h, wi2[...], preferred_element_type=jnp.float32) + bi2[...]
    h = _bn_relu(h, gi2[...], bti2[...])
    t = jnp.dot(h, wg1[...], preferred_element_type=jnp.float32)
    h3a_ref[...] = _dinv_col(dp_ref) * t


def _tc_b_body(h3_ref, p_ref, dp_ref, gg, btg, bg, wnext, h3b_ref):
    dinv = _dinv_col(dp_ref)
    s = dinv * (h3_ref[...] + p_ref[0, :N] + p_ref[1, :N]) + bg[...]
    g = _bn_relu(s, gg[...], btg[...])
    t = jnp.dot(g, wnext[...], preferred_element_type=jnp.float32)
    h3b_ref[...] = dinv * t


def _tc_c_body(h3_ref, p_ref, dp_ref, gg, btg, bg,
               wo1, bo1, go1, bto1, wo2, bo2, go2, bto2, wo3, bo3, out_ref):
    dinv = _dinv_col(dp_ref)
    s = dinv * (h3_ref[...] + p_ref[0, :N] + p_ref[1, :N]) + bg[...]
    g = _bn_relu(s, gg[...], btg[...])
    o = _bn_relu(jnp.dot(g, wo1[...], preferred_element_type=jnp.float32) + bo1[...],
                 go1[...], bto1[...])
    o = _bn_relu(jnp.dot(o, wo2[...], preferred_element_type=jnp.float32) + bo2[...],
                 go2[...], bto2[...])
    out_ref[...] = jnp.dot(o, wo3[...], preferred_element_type=jnp.float32) + bo3[...]


def _tc_call(body, out_shape, *args):
    return pl.pallas_call(
        body,
        out_shape=jax.ShapeDtypeStruct(out_shape, jnp.float32),
    )(*args)


def kernel(x, edge_index, edge_weight,
           Wi1, bi1, gi1, bti1, Wi2, bi2, gi2, bti2,
           Wg1, bg1, gg1, btg1, Wg2, bg2, gg2, btg2,
           Wo1, bo1, go1, bto1, Wo2, bo2, go2, bto2, Wo3, bo3):
    pad = E_PAD - E
    rowp = jnp.concatenate([edge_index[0], jnp.zeros((pad,), jnp.int32)])
    colp = jnp.concatenate([edge_index[1], jnp.zeros((pad,), jnp.int32)])
    ewp = jnp.concatenate([edge_weight, jnp.zeros((pad,), jnp.float32)])

    nblk = E_PAD // EB
    pk = jnp.concatenate([
        rowp.reshape(nblk, 1, EB),
        colp.reshape(nblk, 1, EB),
        lax.bitcast_convert_type(ewp, jnp.int32).reshape(nblk, 1, EB),
        jnp.zeros((nblk, 5, EB), jnp.int32),
    ], axis=1)

    dp = _deg(colp, ewp)

    h3a = _tc_call(_tc_a_body, (N, D),
                   x, dp, Wi1, bi1, gi1, bti1, Wi2, bi2, gi2, bti2, Wg1)
    p1 = _spmm(h3a, pk)
    h3b = _tc_call(_tc_b_body, (N, D), h3a, p1, dp, gg1, btg1, bg1, Wg2)
    p2 = _spmm(h3b, pk)
    out = _tc_call(_tc_c_body, (N, 4), h3b, p2, dp, gg2, btg2, bg2,
                   Wo1, bo1, go1, bto1, Wo2, bo2, go2, bto2, Wo3, bo3)
    return out


# speedup vs baseline: 21.3009x; 2.3040x over previous
import functools

import jax
import jax.numpy as jnp
from jax import lax
from jax.experimental import pallas as pl
from jax.experimental.pallas import tpu as pltpu
from jax.experimental.pallas import tpu_sc as plsc

N = 10000
E = 320000
D = 128
L = 16
NC = 2
NS = 16
NW = NC * NS
EB = 128
NBATCH = 80
NB_A = 80
NB_B = 80
ET = EB * NBATCH
E_PAD = ET * NW
N_PAD = 10240
ROWS_PER_TILE = N_PAD // NS
ZROWS = 128

_MESH = plsc.VectorSubcoreMesh(core_axis_name="c", subcore_axis_name="s")


def _wid():
    return lax.axis_index("c") * NS + lax.axis_index("s")


def _deg_body(col_hbm, ew_hbm, out_hbm, colv, ewv, acc):
    wid = _wid()

    def zero(i, _):
        acc[pl.ds(i * L, L)] = jnp.zeros((L,), jnp.float32)
        return _

    lax.fori_loop(0, N_PAD // L, zero, None)
    base = wid * ET
    pltpu.sync_copy(col_hbm.at[pl.ds(base, ET)], colv)
    pltpu.sync_copy(ew_hbm.at[pl.ds(base, ET)], ewv)

    def body(i, _):
        idx = colv[pl.ds(i * L, L)]
        w = ewv[pl.ds(i * L, L)]
        plsc.addupdate_scatter(acc, [idx], w)
        return _

    lax.fori_loop(0, ET // L, body, None)
    pltpu.sync_copy(acc, out_hbm.at[wid])


@jax.jit
def _deg(col, ew):
    return pl.kernel(
        _deg_body,
        out_type=jax.ShapeDtypeStruct((NW, N_PAD), jnp.float32),
        mesh=_MESH,
        scratch_types=[
            pltpu.VMEM((ET,), jnp.int32),
            pltpu.VMEM((ET,), jnp.float32),
            pltpu.VMEM((N_PAD,), jnp.float32),
        ],
        compiler_params=pltpu.CompilerParams(needs_layout_passes=False),
    )(col, ew)


def _scale_rows(rowsbuf, ebuf):
    def scale(j, _s):
        ew16 = plsc.bitcast(ebuf[2, pl.ds(j * L, L)], jnp.float32)
        for k in range(L):
            sv = jnp.full((L,), ew16[k], jnp.float32)
            i = j * L + k
            for g in range(D // L):
                rowsbuf[i, pl.ds(g * L, L)] = rowsbuf[i, pl.ds(g * L, L)] * sv
        return _s

    lax.fori_loop(0, EB // L, scale, None)


def _spmm_body(h3_hbm, pk_hbm, out_hbm,
               ebuf0, ebuf1, ebuf2, ebuf3, rows0, rows1, acc,
               ge0, ge1, se0, se1, ee0, ee1, ee2, ee3):
    cid = lax.axis_index("c")
    sid = lax.axis_index("s")
    nb = jnp.where(cid == 0, NB_A, NB_B)
    wb = jnp.where(cid == 0, sid * NB_A, NS * NB_A + sid * NB_B)
    ebufs = (ebuf0, ebuf1, ebuf2, ebuf3)
    ees = (ee0, ee1, ee2, ee3)

    def zfill(i, _):
        for g in range(D // L):
            rows0[i, pl.ds(g * L, L)] = jnp.zeros((L,), jnp.float32)
        return _

    lax.fori_loop(0, EB, zfill, None)
    for j in range(ROWS_PER_TILE // EB):
        pltpu.sync_copy(rows0, acc.at[pl.ds(sid * ROWS_PER_TILE + j * EB, EB)])

    plsc.subcore_barrier()

    pltpu.async_copy(pk_hbm.at[wb + 0], ebuf0, ee0)
    pltpu.async_copy(pk_hbm.at[wb + 1], ebuf1, ee1)
    pltpu.async_copy(pk_hbm.at[wb + 2], ebuf2, ee2)
    pltpu.make_async_copy(pk_hbm.at[wb + 0], ebuf0, ee0).wait()
    pltpu.async_copy(h3_hbm.at[ebuf0.at[0]], rows0, ge0)

    def half(b, e, rows_a, rows_b, ge_a, ge_b, se_a, se_b):
        ep = (e + 3) % 4
        en = (e + 1) % 4
        pltpu.make_async_copy(h3_hbm.at[ebufs[e].at[0]], rows_a, ge_a).wait()
        _scale_rows(rows_a, ebufs[e])
        pltpu.async_copy(rows_a, acc.at[ebufs[e].at[1]], se_a, add=True)

        @pl.when(b >= 1)
        def _():
            pltpu.make_async_copy(rows_b, acc.at[ebufs[ep].at[1]], se_b).wait()

        @pl.when(b + 3 < nb)
        def _():
            pltpu.async_copy(pk_hbm.at[wb + b + 3], ebufs[ep], ees[ep])

        @pl.when(b + 1 < nb)
        def _():
            pltpu.make_async_copy(pk_hbm.at[wb + b + 1], ebufs[en], ees[en]).wait()
            pltpu.async_copy(h3_hbm.at[ebufs[en].at[0]], rows_b, ge_b)

    def quad(i, _):
        b = 4 * i
        half(b + 0, 0, rows0, rows1, ge0, ge1, se0, se1)
        half(b + 1, 1, rows1, rows0, ge1, ge0, se1, se0)
        half(b + 2, 2, rows0, rows1, ge0, ge1, se0, se1)
        half(b + 3, 3, rows1, rows0, ge1, ge0, se1, se0)
        return _

    lax.fori_loop(0, nb // 4, quad, None)
    pltpu.make_async_copy(rows1, acc.at[ebuf3.at[1]], se1).wait()

    plsc.subcore_barrier()
    pltpu.sync_copy(acc.at[pl.ds(sid * ROWS_PER_TILE, ROWS_PER_TILE)],
                    out_hbm.at[cid, pl.ds(sid * ROWS_PER_TILE, ROWS_PER_TILE)])


@jax.jit
def _spmm(h3, pk):
    return pl.kernel(
        _spmm_body,
        out_type=jax.ShapeDtypeStruct((NC, N_PAD, D), jnp.float32),
        mesh=_MESH,
        scratch_types=(
            [pltpu.VMEM((8, EB), jnp.int32)] * 4
            + [pltpu.VMEM((EB, D), jnp.float32)] * 2
            + [pltpu.VMEM_SHARED((N_PAD, D), jnp.float32)]
            + [pltpu.SemaphoreType.DMA] * 8
        ),
        compiler_params=pltpu.CompilerParams(needs_layout_passes=False),
    )(h3, pk)


def _bn_relu(h, g, b):
    m = jnp.mean(h, axis=0, keepdims=True)
    v = jnp.mean(h * h, axis=0, keepdims=True) - m * m
    return jax.nn.relu((h - m) * lax.rsqrt(v + 1e-5) * g + b)


def _dinv_col(dp_ref):
    deg = jnp.sum(dp_ref[...], axis=0)[:N] + 1.0
    return lax.rsqrt(deg)[:, None]


def _tc_a_body(x_ref, dp_ref, wi1, bi1, gi1, bti1, wi2, bi2, gi2, bti2, wg1,
               h3a_ref):
    h = jnp.dot(x_ref[...], wi1[...], preferred_element_type=jnp.float32) + bi1[...]
    h = _bn_relu(h, gi1[...], bti1[...])
    h = jnp.dot(h, wi2[...], preferred_element_type=jnp.float32) + bi2[...]
    h = _bn_relu(h, gi2[...], bti2[...])
    t = jnp.dot(h, wg1[...], preferred_element_type=jnp.float32)
    h3a_ref[...] = _dinv_col(dp_ref) * t


def _tc_b_body(h3_ref, p_ref, dp_ref, gg, btg, bg, wnext, h3b_ref):
    dinv = _dinv_col(dp_ref)
    s = dinv * (h3_ref[...] + p_ref[0, :N] + p_ref[1, :N]) + bg[...]
    g = _bn_relu(s, gg[...], btg[...])
    t = jnp.dot(g, wnext[...], preferred_element_type=jnp.float32)
    h3b_ref[...] = dinv * t


def _tc_c_body(h3_ref, p_ref, dp_ref, gg, btg, bg,
               wo1, bo1, go1, bto1, wo2, bo2, go2, bto2, wo3, bo3, out_ref):
    dinv = _dinv_col(dp_ref)
    s = dinv * (h3_ref[...] + p_ref[0, :N] + p_ref[1, :N]) + bg[...]
    g = _bn_relu(s, gg[...], btg[...])
    o = _bn_relu(jnp.dot(g, wo1[...], preferred_element_type=jnp.float32) + bo1[...],
                 go1[...], bto1[...])
    o = _bn_relu(jnp.dot(o, wo2[...], preferred_element_type=jnp.float32) + bo2[...],
                 go2[...], bto2[...])
    out_ref[...] = jnp.dot(o, wo3[...], preferred_element_type=jnp.float32) + bo3[...]


def _tc_call(body, out_shape, *args):
    return pl.pallas_call(
        body,
        out_shape=jax.ShapeDtypeStruct(out_shape, jnp.float32),
    )(*args)


def kernel(x, edge_index, edge_weight,
           Wi1, bi1, gi1, bti1, Wi2, bi2, gi2, bti2,
           Wg1, bg1, gg1, btg1, Wg2, bg2, gg2, btg2,
           Wo1, bo1, go1, bto1, Wo2, bo2, go2, bto2, Wo3, bo3):
    pad = E_PAD - E
    pad_idx = jnp.arange(pad, dtype=jnp.int32) % N
    rowp = jnp.concatenate([edge_index[0], pad_idx])
    colp = jnp.concatenate([edge_index[1], pad_idx])
    ewp = jnp.concatenate([edge_weight, jnp.zeros((pad,), jnp.float32)])

    nblk = E_PAD // EB
    pk = jnp.concatenate([
        rowp.reshape(nblk, 1, EB),
        colp.reshape(nblk, 1, EB),
        lax.bitcast_convert_type(ewp, jnp.int32).reshape(nblk, 1, EB),
        jnp.zeros((nblk, 5, EB), jnp.int32),
    ], axis=1)

    dp = _deg(colp, ewp)

    h3a = _tc_call(_tc_a_body, (N, D),
                   x, dp, Wi1, bi1, gi1, bti1, Wi2, bi2, gi2, bti2, Wg1)
    p1 = _spmm(h3a, pk)
    h3b = _tc_call(_tc_b_body, (N, D), h3a, p1, dp, gg1, btg1, bg1, Wg2)
    p2 = _spmm(h3b, pk)
    out = _tc_call(_tc_c_body, (N, 4), h3b, p2, dp, gg2, btg2, bg2,
                   Wo1, bo1, go1, bto1, Wo2, bo2, go2, bto2, Wo3, bo3)
    return out


# speedup vs baseline: 26.2055x; 1.2303x over previous
import functools

import jax
import jax.numpy as jnp
from jax import lax
from jax.experimental import pallas as pl
from jax.experimental.pallas import tpu as pltpu
from jax.experimental.pallas import tpu_sc as plsc

N = 10000
E = 320000
D = 128
L = 16
NC = 2
NS = 16
NW = NC * NS
EB = 128
NBATCH = 80
NB_A = 80
NB_B = 80
ET = EB * NBATCH
E_PAD = ET * NW
N_PAD = 10240
ROWS_PER_TILE = N_PAD // NS
ZROWS = 128

_MESH = plsc.VectorSubcoreMesh(core_axis_name="c", subcore_axis_name="s")


def _wid():
    return lax.axis_index("c") * NS + lax.axis_index("s")


def _deg_body(col_hbm, ew_hbm, out_hbm, colv, ewv, acc):
    wid = _wid()

    def zero(i, _):
        acc[pl.ds(i * L, L)] = jnp.zeros((L,), jnp.float32)
        return _

    lax.fori_loop(0, N_PAD // L, zero, None)
    base = wid * ET
    pltpu.sync_copy(col_hbm.at[pl.ds(base, ET)], colv)
    pltpu.sync_copy(ew_hbm.at[pl.ds(base, ET)], ewv)

    def body(i, _):
        idx = colv[pl.ds(i * L, L)]
        w = ewv[pl.ds(i * L, L)]
        plsc.addupdate_scatter(acc, [idx], w)
        return _

    lax.fori_loop(0, ET // L, body, None)
    pltpu.sync_copy(acc, out_hbm.at[wid])


@jax.jit
def _deg(col, ew):
    return pl.kernel(
        _deg_body,
        out_type=jax.ShapeDtypeStruct((NW, N_PAD), jnp.float32),
        mesh=_MESH,
        scratch_types=[
            pltpu.VMEM((ET,), jnp.int32),
            pltpu.VMEM((ET,), jnp.float32),
            pltpu.VMEM((N_PAD,), jnp.float32),
        ],
        compiler_params=pltpu.CompilerParams(needs_layout_passes=False),
    )(col, ew)


def _scale_rows(rowsbuf, ebuf):
    def scale(j, _s):
        ew16 = plsc.bitcast(ebuf[2, pl.ds(j * L, L)], jnp.float32)
        for k in range(L):
            sv = jnp.full((L,), ew16[k], jnp.float32)
            i = j * L + k
            for g in range(D // L):
                rowsbuf[i, pl.ds(g * L, L)] = rowsbuf[i, pl.ds(g * L, L)] * sv
        return _s

    lax.fori_loop(0, EB // L, scale, None)


def _spmm_body(h3_hbm, pk_hbm, out_hbm,
               ebuf0, ebuf1, ebuf2, ebuf3, rows0, rows1, acc,
               ge0, ge1, se0, se1, ee0, ee1, ee2, ee3):
    cid = lax.axis_index("c")
    sid = lax.axis_index("s")
    nb = jnp.where(cid == 0, NB_A, NB_B)
    wb = jnp.where(cid == 0, sid * NB_A, NS * NB_A + sid * NB_B)
    ebufs = (ebuf0, ebuf1, ebuf2, ebuf3)
    ees = (ee0, ee1, ee2, ee3)

    def zfill(i, _):
        for g in range(D // L):
            rows0[i, pl.ds(g * L, L)] = jnp.zeros((L,), jnp.float32)
        return _

    lax.fori_loop(0, EB, zfill, None)
    for j in range(ROWS_PER_TILE // EB):
        pltpu.sync_copy(rows0, acc.at[pl.ds(sid * ROWS_PER_TILE + j * EB, EB)])

    plsc.subcore_barrier()

    pltpu.async_copy(pk_hbm.at[wb + 0], ebuf0, ee0)
    pltpu.async_copy(pk_hbm.at[wb + 1], ebuf1, ee1)
    pltpu.async_copy(pk_hbm.at[wb + 2], ebuf2, ee2)
    pltpu.make_async_copy(pk_hbm.at[wb + 0], ebuf0, ee0).wait()
    pltpu.async_copy(h3_hbm.at[ebuf0.at[0]], rows0, ge0)

    def half(b, e, rows_a, rows_b, ge_a, ge_b, se_a, se_b):
        ep = (e + 3) % 4
        en = (e + 1) % 4
        pltpu.make_async_copy(h3_hbm.at[ebufs[e].at[0]], rows_a, ge_a).wait()

        @pl.when(b >= 1)
        def _():
            pltpu.make_async_copy(rows_b, acc.at[ebufs[ep].at[1]], se_b).wait()

        @pl.when(b + 3 < nb)
        def _():
            pltpu.async_copy(pk_hbm.at[wb + b + 3], ebufs[ep], ees[ep])

        @pl.when(b + 1 < nb)
        def _():
            pltpu.make_async_copy(pk_hbm.at[wb + b + 1], ebufs[en], ees[en]).wait()
            pltpu.async_copy(h3_hbm.at[ebufs[en].at[0]], rows_b, ge_b)

        _scale_rows(rows_a, ebufs[e])
        pltpu.async_copy(rows_a, acc.at[ebufs[e].at[1]], se_a, add=True)

    def quad(i, _):
        b = 4 * i
        half(b + 0, 0, rows0, rows1, ge0, ge1, se0, se1)
        half(b + 1, 1, rows1, rows0, ge1, ge0, se1, se0)
        half(b + 2, 2, rows0, rows1, ge0, ge1, se0, se1)
        half(b + 3, 3, rows1, rows0, ge1, ge0, se1, se0)
        return _

    lax.fori_loop(0, nb // 4, quad, None)
    pltpu.make_async_copy(rows1, acc.at[ebuf3.at[1]], se1).wait()

    plsc.subcore_barrier()
    pltpu.sync_copy(acc.at[pl.ds(sid * ROWS_PER_TILE, ROWS_PER_TILE)],
                    out_hbm.at[cid, pl.ds(sid * ROWS_PER_TILE, ROWS_PER_TILE)])


@jax.jit
def _spmm(h3, pk):
    return pl.kernel(
        _spmm_body,
        out_type=jax.ShapeDtypeStruct((NC, N_PAD, D), jnp.float32),
        mesh=_MESH,
        scratch_types=(
            [pltpu.VMEM((8, EB), jnp.int32)] * 4
            + [pltpu.VMEM((EB, D), jnp.float32)] * 2
            + [pltpu.VMEM_SHARED((N_PAD, D), jnp.float32)]
            + [pltpu.SemaphoreType.DMA] * 8
        ),
        compiler_params=pltpu.CompilerParams(needs_layout_passes=False),
    )(h3, pk)


def _bn_relu(h, g, b):
    m = jnp.mean(h, axis=0, keepdims=True)
    v = jnp.mean(h * h, axis=0, keepdims=True) - m * m
    return jax.nn.relu((h - m) * lax.rsqrt(v + 1e-5) * g + b)


def _dinv_col(dp_ref):
    deg = jnp.sum(dp_ref[...], axis=0)[:N] + 1.0
    return lax.rsqrt(deg)[:, None]


def _tc_a_body(x_ref, dp_ref, wi1, bi1, gi1, bti1, wi2, bi2, gi2, bti2, wg1,
               h3a_ref):
    h = jnp.dot(x_ref[...], wi1[...], preferred_element_type=jnp.float32) + bi1[...]
    h = _bn_relu(h, gi1[...], bti1[...])
    h = jnp.dot(h, wi2[...], preferred_element_type=jnp.float32) + bi2[...]
    h = _bn_relu(h, gi2[...], bti2[...])
    t = jnp.dot(h, wg1[...], preferred_element_type=jnp.float32)
    h3a_ref[...] = _dinv_col(dp_ref) * t


def _tc_b_body(h3_ref, p_ref, dp_ref, gg, btg, bg, wnext, h3b_ref):
    dinv = _dinv_col(dp_ref)
    s = dinv * (h3_ref[...] + p_ref[0, :N] + p_ref[1, :N]) + bg[...]
    g = _bn_relu(s, gg[...], btg[...])
    t = jnp.dot(g, wnext[...], preferred_element_type=jnp.float32)
    h3b_ref[...] = dinv * t


def _tc_c_body(h3_ref, p_ref, dp_ref, gg, btg, bg,
               wo1, bo1, go1, bto1, wo2, bo2, go2, bto2, wo3, bo3, out_ref):
    dinv = _dinv_col(dp_ref)
    s = dinv * (h3_ref[...] + p_ref[0, :N] + p_ref[1, :N]) + bg[...]
    g = _bn_relu(s, gg[...], btg[...])
    o = _bn_relu(jnp.dot(g, wo1[...], preferred_element_type=jnp.float32) + bo1[...],
                 go1[...], bto1[...])
    o = _bn_relu(jnp.dot(o, wo2[...], preferred_element_type=jnp.float32) + bo2[...],
                 go2[...], bto2[...])
    out_ref[...] = jnp.dot(o, wo3[...], preferred_element_type=jnp.float32) + bo3[...]


def _tc_call(body, out_shape, *args):
    return pl.pallas_call(
        body,
        out_shape=jax.ShapeDtypeStruct(out_shape, jnp.float32),
    )(*args)


def kernel(x, edge_index, edge_weight,
           Wi1, bi1, gi1, bti1, Wi2, bi2, gi2, bti2,
           Wg1, bg1, gg1, btg1, Wg2, bg2, gg2, btg2,
           Wo1, bo1, go1, bto1, Wo2, bo2, go2, bto2, Wo3, bo3):
    pad = E_PAD - E
    pad_idx = jnp.arange(pad, dtype=jnp.int32) % N
    rowp = jnp.concatenate([edge_index[0], pad_idx])
    colp = jnp.concatenate([edge_index[1], pad_idx])
    ewp = jnp.concatenate([edge_weight, jnp.zeros((pad,), jnp.float32)])

    nblk = E_PAD // EB
    pk = jnp.concatenate([
        rowp.reshape(nblk, 1, EB),
        colp.reshape(nblk, 1, EB),
        lax.bitcast_convert_type(ewp, jnp.int32).reshape(nblk, 1, EB),
        jnp.zeros((nblk, 5, EB), jnp.int32),
    ], axis=1)

    dp = _deg(colp, ewp)

    h3a = _tc_call(_tc_a_body, (N, D),
                   x, dp, Wi1, bi1, gi1, bti1, Wi2, bi2, gi2, bti2, Wg1)
    p1 = _spmm(h3a, pk)
    h3b = _tc_call(_tc_b_body, (N, D), h3a, p1, dp, gg1, btg1, bg1, Wg2)
    p2 = _spmm(h3b, pk)
    out = _tc_call(_tc_c_body, (N, 4), h3b, p2, dp, gg2, btg2, bg2,
                   Wo1, bo1, go1, bto1, Wo2, bo2, go2, bto2, Wo3, bo3)
    return out
